# Initial kernel scaffold; baseline (speedup 1.0000x reference)
import jax
import jax.numpy as jnp
from jax.experimental import pallas as pl


def kernel(hidden_states, attention_mask, Wsc, bsc, Wsu, bsu, cWq, cbq, cWk, cbk, cWv, cbv, cWo, cbo, uWq, ubq, uWk, ubk, uWv, ubv, uWo, ubo):
    raise NotImplementedError("write your pallas kernel here")


# speedup vs baseline: 31.9814x; 31.9814x over previous
import functools

import jax
import jax.numpy as jnp
from jax.experimental import pallas as pl
from jax.experimental.pallas import tpu as pltpu

NH = 12
CAP_FRAC = 1.25


def _transpose_col(v, eye):
    return jax.lax.dot_general(v, eye, (((0,), (0,)), ((), ())),
                               preferred_element_type=jnp.float32,
                               precision=jax.lax.Precision.HIGHEST)


def _routing_kernel(x_ref, wsc_ref, bsc_ref, wsu_ref, bsu_ref,
                    rc_ref, ru_ref, dd_ref, acc_ref, *, cap):
    b = pl.program_id(0)
    nb = pl.num_programs(0)
    seq = x_ref.shape[1]
    xm = jnp.sum(x_ref[0], axis=0, keepdims=True) * (1.0 / seq)
    acc_ref[pl.ds(b, 1), :] = xm

    @pl.when(b == nb - 1)
    def _finalize():
        xall = acc_ref[...]
        bsz = xall.shape[0]
        eye = (jax.lax.broadcasted_iota(jnp.int32, (bsz, bsz), 0) ==
               jax.lax.broadcasted_iota(jnp.int32, (bsz, bsz), 1)
               ).astype(jnp.float32)

        def route(w_ref, b_ref):
            logits = jnp.dot(xall, w_ref[...],
                             preferred_element_type=jnp.float32,
                             precision=jax.lax.Precision.HIGHEST) + b_ref[...]
            p = jax.nn.softmax(logits, axis=-1)
            pmax = jnp.max(p, axis=-1, keepdims=True)
            ne = logits.shape[1]
            col = jax.lax.broadcasted_iota(jnp.int32, logits.shape, 1)
            r = jnp.min(jnp.where(p >= pmax, col, ne), axis=-1,
                        keepdims=True)
            return r.astype(jnp.float32), pmax

        rc, pmc = route(wsc_ref, bsc_ref)
        ru, _ = route(wsu_ref, bsu_ref)

        rc_row = _transpose_col(rc, eye)
        pm_row = _transpose_col(pmc, eye)
        idx_col = jax.lax.broadcasted_iota(jnp.int32, (bsz, bsz), 0)
        idx_row = jax.lax.broadcasted_iota(jnp.int32, (bsz, bsz), 1)
        same = rc_row == rc
        beats = (pm_row > pmc) | ((pm_row == pmc) & (idx_row < idx_col))
        rank = jnp.sum(jnp.where(same & beats, 1.0, 0.0), axis=-1,
                       keepdims=True)
        dropped = jnp.where(rank >= cap, 1.0, 0.0)

        rc_ref[...] = rc_row.astype(jnp.int32)
        ru_ref[...] = _transpose_col(ru, eye).astype(jnp.int32)
        dd_ref[...] = _transpose_col(dropped, eye).astype(jnp.int32)


def _expert_kernel(rc_ref, ru_ref, dd_ref,
                   x_ref, mask_ref,
                   cwq, cbq, cwk, cbk, cwv, cbv, cwo, cbo,
                   uwq, ubq, uwk, ubk, uwv, ubv, uwo, ubo,
                   o_ref, ctx_ref):
    b = pl.program_id(0)
    x = x_ref[0]
    seq, hid = x.shape
    dh = hid // NH
    scale = 1.0 / (dh ** 0.5)
    ext = (1.0 - mask_ref[0].astype(jnp.float32)) * -10000.0

    def mha(wq, bq, wk, bk, wv, bv, wo, bo):
        q = jnp.dot(x, wq[0], preferred_element_type=jnp.float32) + bq[0]
        k = jnp.dot(x, wk[0], preferred_element_type=jnp.float32) + bk[0]
        v = jnp.dot(x, wv[0], preferred_element_type=jnp.float32) + bv[0]
        for h in range(NH):
            sl = slice(h * dh, (h + 1) * dh)
            qh, kh, vh = q[:, sl], k[:, sl], v[:, sl]
            s = jax.lax.dot_general(qh, kh, (((1,), (1,)), ((), ())),
                                    preferred_element_type=jnp.float32)
            s = s * scale + ext
            s = s - jnp.max(s, axis=-1, keepdims=True)
            e = jnp.exp(s)
            p = e / jnp.sum(e, axis=-1, keepdims=True)
            ctx_ref[:, sl] = jnp.dot(p, vh, preferred_element_type=jnp.float32)
        return jnp.dot(ctx_ref[...], wo[0],
                       preferred_element_type=jnp.float32) + bo[0]

    o_ref[0] = mha(cwq, cbq, cwk, cbk, cwv, cbv, cwo, cbo)

    @pl.when(dd_ref[b] == 1)
    def _tail():
        o_ref[0] = o_ref[0] + mha(uwq, ubq, uwk, ubk, uwv, ubv, uwo, ubo)


def kernel(hidden_states, attention_mask, Wsc, bsc, Wsu, bsu,
           cWq, cbq, cWk, cbk, cWv, cbv, cWo, cbo,
           uWq, ubq, uWk, ubk, uWv, ubv, uWo, ubo):
    x = hidden_states
    B, S, H = x.shape
    EC = Wsc.shape[1]
    EU = Wsu.shape[1]
    cap = int(CAP_FRAC * B / EC)

    rc, ru, dd = pl.pallas_call(
        functools.partial(_routing_kernel, cap=cap),
        grid=(B,),
        in_specs=[
            pl.BlockSpec((1, S, H), lambda b: (b, 0, 0)),
            pl.BlockSpec((H, EC), lambda b: (0, 0)),
            pl.BlockSpec((1, EC), lambda b: (0, 0)),
            pl.BlockSpec((H, EU), lambda b: (0, 0)),
            pl.BlockSpec((1, EU), lambda b: (0, 0)),
        ],
        out_specs=[
            pl.BlockSpec((1, B), lambda b: (0, 0)),
            pl.BlockSpec((1, B), lambda b: (0, 0)),
            pl.BlockSpec((1, B), lambda b: (0, 0)),
        ],
        out_shape=[
            jax.ShapeDtypeStruct((1, B), jnp.int32),
            jax.ShapeDtypeStruct((1, B), jnp.int32),
            jax.ShapeDtypeStruct((1, B), jnp.int32),
        ],
        scratch_shapes=[pltpu.VMEM((B, H), jnp.float32)],
    )(x, Wsc, bsc.reshape(1, EC), Wsu, bsu.reshape(1, EU))
    rc = rc.reshape(B)
    ru = ru.reshape(B)
    dd = dd.reshape(B)

    mask2 = attention_mask.reshape(B, 1, S)

    def wspec(ne):
        del ne
        return pl.BlockSpec((1, H, H), lambda b, rcs, rus, dds: (rcs[b], 0, 0))

    def bspec(ne):
        del ne
        return pl.BlockSpec((1, 1, H), lambda b, rcs, rus, dds: (rcs[b], 0, 0))

    def uwspec(ne):
        del ne
        return pl.BlockSpec(
            (1, H, H),
            lambda b, rcs, rus, dds: (jnp.where(dds[b] == 1, rus[b], 0), 0, 0))

    def ubspec(ne):
        del ne
        return pl.BlockSpec(
            (1, 1, H),
            lambda b, rcs, rus, dds: (jnp.where(dds[b] == 1, rus[b], 0), 0, 0))

    grid_spec = pltpu.PrefetchScalarGridSpec(
        num_scalar_prefetch=3,
        grid=(B,),
        in_specs=[
            pl.BlockSpec((1, S, H), lambda b, rcs, rus, dds: (b, 0, 0)),
            pl.BlockSpec((1, 1, S), lambda b, rcs, rus, dds: (b, 0, 0)),
            wspec(EC), bspec(EC), wspec(EC), bspec(EC),
            wspec(EC), bspec(EC), wspec(EC), bspec(EC),
            uwspec(EU), ubspec(EU), uwspec(EU), ubspec(EU),
            uwspec(EU), ubspec(EU), uwspec(EU), ubspec(EU),
        ],
        out_specs=pl.BlockSpec((1, S, H), lambda b, rcs, rus, dds: (b, 0, 0)),
        scratch_shapes=[pltpu.VMEM((S, H), jnp.float32)],
    )

    out = pl.pallas_call(
        _expert_kernel,
        grid_spec=grid_spec,
        out_shape=jax.ShapeDtypeStruct((B, S, H), jnp.float32),
    )(rc, ru, dd, x, mask2,
      cWq, cbq.reshape(EC, 1, H), cWk, cbk.reshape(EC, 1, H),
      cWv, cbv.reshape(EC, 1, H), cWo, cbo.reshape(EC, 1, H),
      uWq, ubq.reshape(EU, 1, H), uWk, ubk.reshape(EU, 1, H),
      uWv, ubv.reshape(EU, 1, H), uWo, ubo.reshape(EU, 1, H))
    return out
